# baseline (speedup 1.0000x reference)
import functools

import jax
import jax.numpy as jnp
from jax import lax
from jax.experimental import pallas as pl
from jax.experimental.pallas import tpu as pltpu
from jax.experimental.pallas import tpu_sc as plsc

SCALE = 2
NLANES = 16
NCORES = 2
NSUBCORES = 16
NWORKERS = NCORES * NSUBCORES
ROW_UNROLL = 8


def _sc_upsample(n, h, w):
    oh, ow = SCALE * h, SCALE * w
    cpw = n // NWORKERS
    npairs = cpw // 2
    assert cpw % 2 == 0 and npairs >= 2 and h % ROW_UNROLL == 0
    mesh = plsc.VectorSubcoreMesh(core_axis_name="c", subcore_axis_name="s")

    @functools.partial(
        pl.kernel,
        mesh=mesh,
        out_type=jax.ShapeDtypeStruct((n, oh, ow), jnp.float32),
        scratch_types=[
            pltpu.VMEM((h, w), jnp.float32),
            pltpu.VMEM((h, w), jnp.float32),
            pltpu.VMEM((oh, ow), jnp.float32),
            pltpu.VMEM((oh, ow), jnp.float32),
            pltpu.SemaphoreType.DMA,
            pltpu.SemaphoreType.DMA,
            pltpu.SemaphoreType.DMA,
            pltpu.SemaphoreType.DMA,
        ],
        compiler_params=pltpu.CompilerParams(needs_layout_passes=False),
    )
    def k(x_hbm, out_hbm, xb0, xb1, ob0, ob1, si0, si1, so0, so1):
        wid = lax.axis_index("s") * NCORES + lax.axis_index("c")
        base = wid * cpw
        zeros16 = jnp.zeros((NLANES,), jnp.float32)

        def zero_blk(r, _):
            for g in range(ow // NLANES):
                ob0[r, pl.ds(g * NLANES, NLANES)] = zeros16
                ob1[r, pl.ds(g * NLANES, NLANES)] = zeros16
            return 0

        lax.fori_loop(0, oh, zero_blk, 0)

        lane = lax.iota(jnp.int32, NLANES)
        col_idx = [2 * lane + (2 * NLANES * g + 1) for g in range(w // NLANES)]

        def scatter_channel(xb, ob):
            def do_rows(r, _):
                for u in range(ROW_UNROLL):
                    hh = r * ROW_UNROLL + u
                    row_idx = jnp.full((NLANES,), 2 * hh + 1, jnp.int32)
                    for g in range(w // NLANES):
                        vals = xb[hh, pl.ds(g * NLANES, NLANES)]
                        plsc.store_scatter(ob, [row_idx, col_idx[g]], vals)
                return 0

            lax.fori_loop(0, h // ROW_UNROLL, do_rows, 0)

        def in_start(ch, xb, sem):
            pltpu.async_copy(x_hbm.at[ch], xb, sem)

        def in_wait(xb, sem):
            pltpu.make_async_copy(x_hbm.at[base], xb, sem).wait()

        def out_start(ob, ch, sem):
            pltpu.async_copy(ob, out_hbm.at[ch], sem)

        def out_wait(ob, sem):
            pltpu.make_async_copy(ob, out_hbm.at[base], sem).wait()

        in_start(base, xb0, si0)
        in_start(base + 1, xb1, si1)
        in_wait(xb0, si0)
        scatter_channel(xb0, ob0)
        out_start(ob0, base, so0)
        in_start(base + 2, xb0, si0)
        in_wait(xb1, si1)
        scatter_channel(xb1, ob1)
        out_start(ob1, base + 1, so1)
        in_start(base + 3, xb1, si1)

        def pair_body(p, _):
            ch0 = base + 2 * p
            in_wait(xb0, si0)
            out_wait(ob0, so0)
            scatter_channel(xb0, ob0)
            out_start(ob0, ch0, so0)
            in_start(ch0 + 2, xb0, si0)
            in_wait(xb1, si1)
            out_wait(ob1, so1)
            scatter_channel(xb1, ob1)
            out_start(ob1, ch0 + 1, so1)
            in_start(ch0 + 3, xb1, si1)
            return 0

        lax.fori_loop(1, npairs - 1, pair_body, 0)

        ch0 = base + cpw - 2
        in_wait(xb0, si0)
        out_wait(ob0, so0)
        scatter_channel(xb0, ob0)
        out_start(ob0, ch0, so0)
        in_wait(xb1, si1)
        out_wait(ob1, so1)
        scatter_channel(xb1, ob1)
        out_start(ob1, ch0 + 1, so1)
        out_wait(ob0, so0)
        out_wait(ob1, so1)

    return k


def kernel(x):
    B, I, C, H, W = x.shape
    n = B * I * C
    xf = x.reshape(n, H, W)
    out = _sc_upsample(n, H, W)(xf)
    return out.reshape(B, I, C, SCALE * H, SCALE * W)


# speedup vs baseline: 1.0217x; 1.0217x over previous
import functools

import jax
import jax.numpy as jnp
from jax import lax
from jax.experimental import pallas as pl
from jax.experimental.pallas import tpu as pltpu
from jax.experimental.pallas import tpu_sc as plsc

SCALE = 2
NLANES = 16
NCORES = 2
NSUBCORES = 16
NWORKERS = NCORES * NSUBCORES
ROW_UNROLL = 8


def _sc_upsample(n, h, w):
    oh, ow = SCALE * h, SCALE * w
    cpw = n // NWORKERS
    npairs = cpw // 2
    assert cpw % 2 == 0 and npairs >= 2 and h % ROW_UNROLL == 0
    mesh = plsc.VectorSubcoreMesh(core_axis_name="c", subcore_axis_name="s")

    @functools.partial(
        pl.kernel,
        mesh=mesh,
        out_type=jax.ShapeDtypeStruct((n, oh, ow), jnp.float32),
        scratch_types=[
            pltpu.VMEM((h, w), jnp.float32),
            pltpu.VMEM((h, w), jnp.float32),
            pltpu.VMEM((oh, ow), jnp.float32),
            pltpu.VMEM((oh, ow), jnp.float32),
            pltpu.SemaphoreType.DMA,
            pltpu.SemaphoreType.DMA,
            pltpu.SemaphoreType.DMA,
            pltpu.SemaphoreType.DMA,
        ],
        compiler_params=pltpu.CompilerParams(needs_layout_passes=False),
    )
    def k(x_hbm, out_hbm, xb0, xb1, ob0, ob1, si0, si1, so0, so1):
        wid = lax.axis_index("s") * NCORES + lax.axis_index("c")
        base = wid * cpw
        zeros16 = jnp.zeros((NLANES,), jnp.float32)

        pltpu.async_copy(x_hbm.at[base], xb0, si0)
        pltpu.async_copy(x_hbm.at[base + 1], xb1, si1)

        def zero_blk(r, _):
            for g in range(ow // NLANES):
                ob0[r, pl.ds(g * NLANES, NLANES)] = zeros16
                ob1[r, pl.ds(g * NLANES, NLANES)] = zeros16
            return 0

        lax.fori_loop(0, oh, zero_blk, 0)

        lane = lax.iota(jnp.int32, NLANES)
        col_idx = [2 * lane + (2 * NLANES * g + 1) for g in range(w // NLANES)]

        def scatter_channel(xb, ob):
            def do_rows(r, _):
                for u in range(ROW_UNROLL):
                    hh = r * ROW_UNROLL + u
                    row_idx = jnp.full((NLANES,), 2 * hh + 1, jnp.int32)
                    for g in range(w // NLANES):
                        vals = xb[hh, pl.ds(g * NLANES, NLANES)]
                        plsc.store_scatter(ob, [row_idx, col_idx[g]], vals)
                return 0

            lax.fori_loop(0, h // ROW_UNROLL, do_rows, 0)

        def in_start(ch, xb, sem):
            pltpu.async_copy(x_hbm.at[ch], xb, sem)

        def in_wait(xb, sem):
            pltpu.make_async_copy(x_hbm.at[base], xb, sem).wait()

        def out_start(ob, ch, sem):
            pltpu.async_copy(ob, out_hbm.at[ch], sem)

        def out_wait(ob, sem):
            pltpu.make_async_copy(ob, out_hbm.at[base], sem).wait()

        in_wait(xb0, si0)
        scatter_channel(xb0, ob0)
        out_start(ob0, base, so0)
        in_start(base + 2, xb0, si0)
        in_wait(xb1, si1)
        scatter_channel(xb1, ob1)
        out_start(ob1, base + 1, so1)
        in_start(base + 3, xb1, si1)

        def pair_body(p, _):
            ch0 = base + 2 * p
            in_wait(xb0, si0)
            out_wait(ob0, so0)
            scatter_channel(xb0, ob0)
            out_start(ob0, ch0, so0)
            in_start(ch0 + 2, xb0, si0)
            in_wait(xb1, si1)
            out_wait(ob1, so1)
            scatter_channel(xb1, ob1)
            out_start(ob1, ch0 + 1, so1)
            in_start(ch0 + 3, xb1, si1)
            return 0

        lax.fori_loop(1, npairs - 1, pair_body, 0)

        ch0 = base + cpw - 2
        in_wait(xb0, si0)
        out_wait(ob0, so0)
        scatter_channel(xb0, ob0)
        out_start(ob0, ch0, so0)
        in_wait(xb1, si1)
        out_wait(ob1, so1)
        scatter_channel(xb1, ob1)
        out_start(ob1, ch0 + 1, so1)
        out_wait(ob0, so0)
        out_wait(ob1, so1)

    return k


def kernel(x):
    B, I, C, H, W = x.shape
    n = B * I * C
    xf = x.reshape(n, H, W)
    out = _sc_upsample(n, H, W)(xf)
    return out.reshape(B, I, C, SCALE * H, SCALE * W)


# speedup vs baseline: 1.0237x; 1.0020x over previous
import functools

import jax
import jax.numpy as jnp
from jax import lax
from jax.experimental import pallas as pl
from jax.experimental.pallas import tpu as pltpu
from jax.experimental.pallas import tpu_sc as plsc

SCALE = 2
NLANES = 16
NCORES = 2
NSUBCORES = 16
NWORKERS = NCORES * NSUBCORES
ROW_UNROLL = 8


def _sc_upsample(n, h, w):
    oh, ow = SCALE * h, SCALE * w
    cpw = n // NWORKERS
    npairs = cpw // 2
    assert cpw % 2 == 0 and npairs >= 2 and h % ROW_UNROLL == 0
    mesh = plsc.VectorSubcoreMesh(core_axis_name="c", subcore_axis_name="s")

    @functools.partial(
        pl.kernel,
        mesh=mesh,
        out_type=jax.ShapeDtypeStruct((n, oh, ow), jnp.float32),
        scratch_types=[
            pltpu.VMEM((h, w), jnp.float32),
            pltpu.VMEM((h, w), jnp.float32),
            pltpu.VMEM((oh, ow), jnp.float32),
            pltpu.VMEM((oh, ow), jnp.float32),
            pltpu.SemaphoreType.DMA,
            pltpu.SemaphoreType.DMA,
            pltpu.SemaphoreType.DMA,
            pltpu.SemaphoreType.DMA,
        ],
        compiler_params=pltpu.CompilerParams(needs_layout_passes=False),
    )
    def k(x_hbm, out_hbm, xb0, xb1, ob0, ob1, si0, si1, so0, so1):
        wid = lax.axis_index("s") * NCORES + lax.axis_index("c")
        base = wid * cpw
        zeros16 = jnp.zeros((NLANES,), jnp.float32)

        pltpu.async_copy(x_hbm.at[base], xb0, si0)
        pltpu.async_copy(x_hbm.at[base + 1], xb1, si1)

        def zero_buf(ob):
            def zero_blk(r, _):
                for g in range(ow // NLANES):
                    ob[r, pl.ds(g * NLANES, NLANES)] = zeros16
                return 0

            lax.fori_loop(0, oh, zero_blk, 0)

        zero_buf(ob0)

        lane = lax.iota(jnp.int32, NLANES)
        col_idx = [2 * lane + (2 * NLANES * g + 1) for g in range(w // NLANES)]

        def scatter_channel(xb, ob):
            def do_rows(r, _):
                for u in range(ROW_UNROLL):
                    hh = r * ROW_UNROLL + u
                    row_idx = jnp.full((NLANES,), 2 * hh + 1, jnp.int32)
                    for g in range(w // NLANES):
                        vals = xb[hh, pl.ds(g * NLANES, NLANES)]
                        plsc.store_scatter(ob, [row_idx, col_idx[g]], vals)
                return 0

            lax.fori_loop(0, h // ROW_UNROLL, do_rows, 0)

        def in_start(ch, xb, sem):
            pltpu.async_copy(x_hbm.at[ch], xb, sem)

        def in_wait(xb, sem):
            pltpu.make_async_copy(x_hbm.at[base], xb, sem).wait()

        def out_start(ob, ch, sem):
            pltpu.async_copy(ob, out_hbm.at[ch], sem)

        def out_wait(ob, sem):
            pltpu.make_async_copy(ob, out_hbm.at[base], sem).wait()

        in_wait(xb0, si0)
        scatter_channel(xb0, ob0)
        out_start(ob0, base, so0)
        in_start(base + 2, xb0, si0)
        zero_buf(ob1)
        in_wait(xb1, si1)
        scatter_channel(xb1, ob1)
        out_start(ob1, base + 1, so1)
        in_start(base + 3, xb1, si1)

        def pair_body(p, _):
            ch0 = base + 2 * p
            in_wait(xb0, si0)
            out_wait(ob0, so0)
            scatter_channel(xb0, ob0)
            out_start(ob0, ch0, so0)
            in_start(ch0 + 2, xb0, si0)
            in_wait(xb1, si1)
            out_wait(ob1, so1)
            scatter_channel(xb1, ob1)
            out_start(ob1, ch0 + 1, so1)
            in_start(ch0 + 3, xb1, si1)
            return 0

        lax.fori_loop(1, npairs - 1, pair_body, 0)

        ch0 = base + cpw - 2
        in_wait(xb0, si0)
        out_wait(ob0, so0)
        scatter_channel(xb0, ob0)
        out_start(ob0, ch0, so0)
        in_wait(xb1, si1)
        out_wait(ob1, so1)
        scatter_channel(xb1, ob1)
        out_start(ob1, ch0 + 1, so1)
        out_wait(ob0, so0)
        out_wait(ob1, so1)

    return k


def kernel(x):
    B, I, C, H, W = x.shape
    n = B * I * C
    xf = x.reshape(n, H, W)
    out = _sc_upsample(n, H, W)(xf)
    return out.reshape(B, I, C, SCALE * H, SCALE * W)
